# Initial kernel scaffold; baseline (speedup 1.0000x reference)
import functools

import jax
import jax.numpy as jnp
from jax import lax
from jax.experimental import pallas as pl
from jax.experimental.pallas import tpu as pltpu
from jax.experimental.pallas import tpu_sc as plsc

N = 320000
D = 128
S = 10000
CNT_W = 16

NC = 2
NS = 16
ROWS_PER_TILE = N // (NC * NS)
CHUNK = 128
N_FULL = ROWS_PER_TILE // CHUNK
TAIL = ROWS_PER_TILE - N_FULL * CHUNK
SEG_PER_TILE = S // NS
ZROWS = 125


def _fill_rows(buf, nrows, ncols, val):
    for c in range(ncols // 16):
        buf[0, pl.ds(c * 16, 16)] = jnp.full((16,), val, jnp.float32)
    filled = 1
    while filled < nrows:
        n = min(filled, nrows - filled)
        pltpu.sync_copy(buf.at[pl.ds(0, n), :], buf.at[pl.ds(filled, n), :])
        filled += n


def _sc_partial_sums(data, seg_ids):
    mesh = plsc.VectorSubcoreMesh(core_axis_name="c", subcore_axis_name="s")

    @functools.partial(
        pl.kernel,
        out_type=(
            jax.ShapeDtypeStruct((NC, S, D), jnp.float32),
            jax.ShapeDtypeStruct((NC, S, CNT_W), jnp.float32),
        ),
        mesh=mesh,
        scratch_types=dict(
            acc_sh=pltpu.VMEM_SHARED((S, D), jnp.float32),
            cnt_sh=pltpu.VMEM_SHARED((S, CNT_W), jnp.float32),
            idx_v=pltpu.VMEM((CHUNK,), jnp.int32),
            dat_v=pltpu.VMEM((CHUNK, D), jnp.float32),
            idx_t=pltpu.VMEM((TAIL,), jnp.int32),
            dat_t=pltpu.VMEM((TAIL, D), jnp.float32),
            ones_v=pltpu.VMEM((CHUNK, CNT_W), jnp.float32),
            zbuf=pltpu.VMEM((ZROWS, D), jnp.float32),
        ),
    )
    def k(data_hbm, ids_hbm, psum_hbm, pcnt_hbm, *, acc_sh, cnt_sh, idx_v,
          dat_v, idx_t, dat_t, ones_v, zbuf):
        c = lax.axis_index("c")
        s = lax.axis_index("s")
        wid = c * NS + s

        _fill_rows(ones_v, CHUNK, CNT_W, 1.0)
        _fill_rows(zbuf, ZROWS, D, 0.0)

        seg0 = s * SEG_PER_TILE
        for r in range(SEG_PER_TILE // ZROWS):
            pltpu.sync_copy(zbuf, acc_sh.at[pl.ds(seg0 + r * ZROWS, ZROWS), :])
            pltpu.sync_copy(zbuf.at[:, pl.ds(0, CNT_W)],
                            cnt_sh.at[pl.ds(seg0 + r * ZROWS, ZROWS), :])
        plsc.subcore_barrier()

        row0 = wid * ROWS_PER_TILE

        def body(g, carry):
            base = row0 + g * CHUNK
            pltpu.sync_copy(ids_hbm.at[pl.ds(base, CHUNK)], idx_v)
            pltpu.sync_copy(data_hbm.at[pl.ds(base, CHUNK), :], dat_v)
            pltpu.sync_copy(dat_v, acc_sh.at[idx_v], add=True)
            pltpu.sync_copy(ones_v, cnt_sh.at[idx_v], add=True)
            return carry

        lax.fori_loop(0, N_FULL, body, 0)

        base = row0 + N_FULL * CHUNK
        pltpu.sync_copy(ids_hbm.at[pl.ds(base, TAIL)], idx_t)
        pltpu.sync_copy(data_hbm.at[pl.ds(base, TAIL), :], dat_t)
        pltpu.sync_copy(dat_t, acc_sh.at[idx_t], add=True)
        pltpu.sync_copy(ones_v.at[pl.ds(0, TAIL), :], cnt_sh.at[idx_t], add=True)

        plsc.subcore_barrier()

        pltpu.sync_copy(acc_sh.at[pl.ds(seg0, SEG_PER_TILE), :],
                        psum_hbm.at[c, pl.ds(seg0, SEG_PER_TILE), :])
        pltpu.sync_copy(cnt_sh.at[pl.ds(seg0, SEG_PER_TILE), :],
                        pcnt_hbm.at[c, pl.ds(seg0, SEG_PER_TILE), :])

    return k(data, seg_ids)


def _combine_kernel(psum_ref, pcnt_ref, out_ref):
    total = psum_ref[0] + psum_ref[1]
    cnt = pcnt_ref[0] + pcnt_ref[1]
    cnt0 = jnp.maximum(cnt[:, 0:1], 1e-6)
    out_ref[...] = total / cnt0


def _combine(psum, pcnt):
    rows = 1250
    grid = S // rows
    return pl.pallas_call(
        _combine_kernel,
        out_shape=jax.ShapeDtypeStruct((S, D), jnp.float32),
        grid=(grid,),
        in_specs=[
            pl.BlockSpec((NC, rows, D), lambda i: (0, i, 0)),
            pl.BlockSpec((NC, rows, CNT_W), lambda i: (0, i, 0)),
        ],
        out_specs=pl.BlockSpec((rows, D), lambda i: (i, 0)),
    )(psum, pcnt)


def kernel(data, segment_ids, num_segments):
    del num_segments
    psum, pcnt = _sc_partial_sums(data, segment_ids)
    return _combine(psum, pcnt)


# speedup vs baseline: 4.2840x; 4.2840x over previous
import functools

import jax
import jax.numpy as jnp
from jax import lax
from jax.experimental import pallas as pl
from jax.experimental.pallas import tpu as pltpu
from jax.experimental.pallas import tpu_sc as plsc

N = 320000
D = 128
S = 10000

NC = 2
NS = 16
ROWS_PER_TILE = N // (NC * NS)
CHUNK = 64
N_FULL = ROWS_PER_TILE // CHUNK
TAIL = ROWS_PER_TILE - N_FULL * CHUNK
SEG_PER_TILE = 624
SEG_REM = S - NS * SEG_PER_TILE
SP = 10240


def _fill_rows(buf, nrows, ncols, val):
    v = jnp.full((16,), val, jnp.float32)

    def body(i, carry):
        for col in range(ncols // 16):
            buf[i, pl.ds(col * 16, 16)] = v
        return carry

    lax.fori_loop(0, nrows, body, 0)


def _fill_1d(buf, n, val):
    v = jnp.full((16,), val, jnp.float32)

    def body(i, carry):
        buf[pl.ds(i * 16, 16)] = v
        return carry

    lax.fori_loop(0, n // 16, body, 0)


def _sc_partial_sums(data, seg_ids):
    mesh = plsc.VectorSubcoreMesh(core_axis_name="c", subcore_axis_name="s")

    @functools.partial(
        pl.kernel,
        out_type=(
            jax.ShapeDtypeStruct((NC, S, D), jnp.float32),
            jax.ShapeDtypeStruct((NC * SP,), jnp.float32),
        ),
        mesh=mesh,
        scratch_types=dict(
            acc_sh=pltpu.VMEM_SHARED((S, D), jnp.float32),
            cnt_sh=pltpu.VMEM_SHARED((S,), jnp.float32),
            idx_v=pltpu.VMEM((CHUNK,), jnp.int32),
            dat_v=pltpu.VMEM((CHUNK, D), jnp.float32),
            idx_t=pltpu.VMEM((TAIL,), jnp.int32),
            ones_v=pltpu.VMEM((CHUNK,), jnp.float32),
            zcnt=pltpu.VMEM((SEG_PER_TILE,), jnp.float32),
        ),
    )
    def k(data_hbm, ids_hbm, psum_hbm, pcnt_hbm, *, acc_sh, cnt_sh, idx_v,
          dat_v, idx_t, ones_v, zcnt):
        c = lax.axis_index("c")
        s = lax.axis_index("s")
        wid = c * NS + s

        _fill_1d(ones_v, CHUNK, 1.0)
        _fill_1d(zcnt, SEG_PER_TILE, 0.0)
        _fill_rows(dat_v, CHUNK, D, 0.0)

        seg0 = s * SEG_PER_TILE
        zchunks = []
        off = 0
        while off < SEG_PER_TILE:
            zchunks.append((off, min(CHUNK, SEG_PER_TILE - off)))
            off += CHUNK
        for off, nr in zchunks:
            pltpu.sync_copy(dat_v.at[pl.ds(0, nr), :],
                            acc_sh.at[pl.ds(seg0 + off, nr), :])
        pltpu.sync_copy(zcnt, cnt_sh.at[pl.ds(seg0, SEG_PER_TILE)])

        @pl.when(s == NS - 1)
        def _():
            pltpu.sync_copy(dat_v.at[pl.ds(0, SEG_REM), :],
                            acc_sh.at[pl.ds(NS * SEG_PER_TILE, SEG_REM), :])
            pltpu.sync_copy(zcnt.at[pl.ds(0, SEG_REM)],
                            cnt_sh.at[pl.ds(NS * SEG_PER_TILE, SEG_REM)])
        plsc.subcore_barrier()

        row0 = wid * ROWS_PER_TILE

        def body(g, carry):
            base = row0 + g * CHUNK
            pltpu.sync_copy(ids_hbm.at[pl.ds(base, CHUNK)], idx_v)
            pltpu.sync_copy(data_hbm.at[pl.ds(base, CHUNK), :], dat_v)
            pltpu.sync_copy(dat_v, acc_sh.at[idx_v], add=True)
            pltpu.sync_copy(ones_v, cnt_sh.at[idx_v], add=True)
            return carry

        lax.fori_loop(0, N_FULL, body, 0)

        base = row0 + N_FULL * CHUNK
        pltpu.sync_copy(ids_hbm.at[pl.ds(base, TAIL)], idx_t)
        pltpu.sync_copy(data_hbm.at[pl.ds(base, TAIL), :],
                        dat_v.at[pl.ds(0, TAIL), :])
        pltpu.sync_copy(dat_v.at[pl.ds(0, TAIL), :], acc_sh.at[idx_t], add=True)
        pltpu.sync_copy(ones_v.at[pl.ds(0, TAIL)], cnt_sh.at[idx_t], add=True)

        plsc.subcore_barrier()

        pltpu.sync_copy(acc_sh.at[pl.ds(seg0, SEG_PER_TILE), :],
                        psum_hbm.at[c, pl.ds(seg0, SEG_PER_TILE), :])
        pltpu.sync_copy(cnt_sh.at[pl.ds(seg0, SEG_PER_TILE)], zcnt)
        pltpu.sync_copy(zcnt, pcnt_hbm.at[pl.ds(c * SP + seg0, SEG_PER_TILE)])

        @pl.when(s == NS - 1)
        def _():
            base = NS * SEG_PER_TILE
            pltpu.sync_copy(acc_sh.at[pl.ds(base, SEG_REM), :],
                            psum_hbm.at[c, pl.ds(base, SEG_REM), :])
            pltpu.sync_copy(cnt_sh.at[pl.ds(base, SEG_REM)],
                            zcnt.at[pl.ds(0, SEG_REM)])
            pltpu.sync_copy(zcnt.at[pl.ds(0, SEG_REM)],
                            pcnt_hbm.at[pl.ds(c * SP + base, SEG_REM)])

    return k(data, seg_ids)


def _combine_kernel(psum_ref, pcnt_ref, out_ref):
    i = pl.program_id(0)
    rows = out_ref.shape[0]
    total = psum_ref[0] + psum_ref[1]
    cnt = (pcnt_ref[pl.ds(i * rows, rows)]
           + pcnt_ref[pl.ds(SP + i * rows, rows)])
    cnt0 = jnp.maximum(cnt, 1e-6)[:, None]
    out_ref[...] = total / cnt0


def _combine(psum, pcnt):
    rows = 1024
    grid = (S + rows - 1) // rows
    return pl.pallas_call(
        _combine_kernel,
        out_shape=jax.ShapeDtypeStruct((S, D), jnp.float32),
        grid=(grid,),
        in_specs=[
            pl.BlockSpec((NC, rows, D), lambda i: (0, i, 0)),
            pl.BlockSpec((NC * SP,), lambda i: (0,)),
        ],
        out_specs=pl.BlockSpec((rows, D), lambda i: (i, 0)),
    )(psum, pcnt)


def kernel(data, segment_ids, num_segments):
    del num_segments
    psum, pcnt = _sc_partial_sums(data, segment_ids)
    return _combine(psum, pcnt)
